# baseline (speedup 1.0000x reference)
import jax
import jax.numpy as jnp
from jax.experimental import pallas as pl
from jax.experimental.pallas import tpu as pltpu

_TOTAL_TOKENS = 32768
_BATCH = 16
_D_MODEL = 4096
_WAVE = 4
_NWAVES = _BATCH // _WAVE


def _pooler_body(lens_ref, hs_ref, out_hbm, buf, in_sems, out_sems):
    in_copies = []
    running = lens_ref[0]
    for i in range(_BATCH):
        c = pltpu.make_async_copy(
            hs_ref.at[pl.ds(running - 1, 1)], buf.at[pl.ds(i, 1)], in_sems.at[i]
        )
        c.start()
        in_copies.append(c)
        if i + 1 < _BATCH:
            running = running + lens_ref[i + 1]

    out_copies = []
    for w in range(_NWAVES):
        rows = pl.ds(w * _WAVE, _WAVE)
        for c in in_copies[w * _WAVE : (w + 1) * _WAVE]:
            c.wait()
        x = buf[rows, :]
        ss = jnp.sum(x * x, axis=1, keepdims=True)
        scale = jnp.where(ss > 1e-24, jax.lax.rsqrt(ss), 1e12)
        buf[rows, :] = x * scale
        oc = pltpu.make_async_copy(buf.at[rows], out_hbm.at[rows], out_sems.at[w])
        oc.start()
        out_copies.append(oc)
    for oc in out_copies:
        oc.wait()


def kernel(hidden_states, extend_seq_lens):
    return pl.pallas_call(
        _pooler_body,
        out_shape=jax.ShapeDtypeStruct((_BATCH, _D_MODEL), jnp.float32),
        in_specs=[
            pl.BlockSpec(memory_space=pltpu.SMEM),
            pl.BlockSpec(memory_space=pltpu.HBM),
        ],
        out_specs=pl.BlockSpec(memory_space=pltpu.HBM),
        scratch_shapes=[
            pltpu.VMEM((_BATCH, _D_MODEL), jnp.float32),
            pltpu.SemaphoreType.DMA((_BATCH,)),
            pltpu.SemaphoreType.DMA((_NWAVES,)),
        ],
    )(extend_seq_lens, hidden_states)


# speedup vs baseline: 1.1000x; 1.1000x over previous
import jax
import jax.numpy as jnp
from jax.experimental import pallas as pl
from jax.experimental.pallas import tpu as pltpu

_TOTAL_TOKENS = 32768
_BATCH = 16
_D_MODEL = 4096


def _pooler_body(lens_ref, hs_ref, out_ref, sems):
    copies = []
    running = lens_ref[0]
    for i in range(_BATCH):
        c = pltpu.make_async_copy(
            hs_ref.at[pl.ds(running - 1, 1)], out_ref.at[pl.ds(i, 1)], sems.at[i]
        )
        c.start()
        copies.append(c)
        if i + 1 < _BATCH:
            running = running + lens_ref[i + 1]
    for c in copies:
        c.wait()

    x = out_ref[...]
    ss = jnp.sum(x * x, axis=1, keepdims=True)
    scale = jnp.where(ss > 1e-24, jax.lax.rsqrt(ss), 1e12)
    out_ref[...] = x * scale


def kernel(hidden_states, extend_seq_lens):
    return pl.pallas_call(
        _pooler_body,
        out_shape=jax.ShapeDtypeStruct((_BATCH, _D_MODEL), jnp.float32),
        in_specs=[
            pl.BlockSpec(memory_space=pltpu.SMEM),
            pl.BlockSpec(memory_space=pltpu.HBM),
        ],
        out_specs=pl.BlockSpec(memory_space=pltpu.VMEM),
        scratch_shapes=[pltpu.SemaphoreType.DMA((_BATCH,))],
    )(extend_seq_lens, hidden_states)


# speedup vs baseline: 1.1350x; 1.0318x over previous
import jax
import jax.numpy as jnp
from jax.experimental import pallas as pl
from jax.experimental.pallas import tpu as pltpu

_TOTAL_TOKENS = 32768
_BATCH = 16
_D_MODEL = 4096


def _pooler_body(lens_ref, hs_ref, out_ref, sems):
    running = lens_ref[0]
    for i in range(_BATCH):
        pltpu.make_async_copy(
            hs_ref.at[pl.ds(running - 1, 1)], out_ref.at[pl.ds(i, 1)], sems
        ).start()
        if i + 1 < _BATCH:
            running = running + lens_ref[i + 1]
    pltpu.make_async_copy(hs_ref.at[pl.ds(0, _BATCH)], out_ref, sems).wait()

    x = out_ref[...]
    ss = jnp.sum(x * x, axis=1, keepdims=True)
    scale = jnp.where(ss > 1e-24, jax.lax.rsqrt(ss), 1e12)
    out_ref[...] = x * scale


def kernel(hidden_states, extend_seq_lens):
    return pl.pallas_call(
        _pooler_body,
        out_shape=jax.ShapeDtypeStruct((_BATCH, _D_MODEL), jnp.float32),
        in_specs=[
            pl.BlockSpec(memory_space=pltpu.SMEM),
            pl.BlockSpec(memory_space=pltpu.HBM),
        ],
        out_specs=pl.BlockSpec(memory_space=pltpu.VMEM),
        scratch_shapes=[pltpu.SemaphoreType.DMA],
    )(extend_seq_lens, hidden_states)


# speedup vs baseline: 1.1582x; 1.0204x over previous
import jax
import jax.numpy as jnp
from jax.experimental import pallas as pl
from jax.experimental.pallas import tpu as pltpu

_TOTAL_TOKENS = 32768
_BATCH = 16
_D_MODEL = 4096
_HALF = _BATCH // 2


def _pooler_body(lens_ref, hs_ref, out_hbm, buf, in_sem, out_sem):
    running = lens_ref[0]
    for i in range(_BATCH):
        pltpu.make_async_copy(
            hs_ref.at[pl.ds(running - 1, 1)], buf.at[pl.ds(i, 1)], in_sem
        ).start()
        if i + 1 < _BATCH:
            running = running + lens_ref[i + 1]
    pltpu.make_async_copy(hs_ref.at[pl.ds(0, _BATCH)], buf, in_sem).wait()

    for h in range(2):
        rows = pl.ds(h * _HALF, _HALF)
        x = buf[rows, :]
        ss = jnp.sum(x * x, axis=1, keepdims=True)
        scale = jnp.where(ss > 1e-24, jax.lax.rsqrt(ss), 1e12)
        buf[rows, :] = x * scale
        pltpu.make_async_copy(buf.at[rows], out_hbm.at[rows], out_sem).start()
    pltpu.make_async_copy(buf, out_hbm, out_sem).wait()


def kernel(hidden_states, extend_seq_lens):
    return pl.pallas_call(
        _pooler_body,
        out_shape=jax.ShapeDtypeStruct((_BATCH, _D_MODEL), jnp.float32),
        in_specs=[
            pl.BlockSpec(memory_space=pltpu.SMEM),
            pl.BlockSpec(memory_space=pltpu.HBM),
        ],
        out_specs=pl.BlockSpec(memory_space=pltpu.HBM),
        scratch_shapes=[
            pltpu.VMEM((_BATCH, _D_MODEL), jnp.float32),
            pltpu.SemaphoreType.DMA,
            pltpu.SemaphoreType.DMA,
        ],
    )(extend_seq_lens, hidden_states)
